# baseline (speedup 1.0000x reference)
import functools

import jax
import jax.numpy as jnp
from jax import lax
from jax.experimental import pallas as pl
from jax.experimental.pallas import tpu as pltpu
from jax.experimental.pallas import tpu_sc as plsc

N = 10000
E = 320000
F_IN = 128
P = 12
NC = 2
NS = 16
CHUNK = 80
DEG_ITERS = E // (NC * NS) // CHUNK
AGG_ITERS = E // NS // CHUNK
NPT = N // NS
FH = 144

_f32 = jnp.float32


def _deg_body(dst_hbm, zeros_hbm, deg_out, didx, ones_v, acc, sem):
    cid = lax.axis_index("c")
    sid = lax.axis_index("s")
    wid = sid * NC + cid
    for i in range(CHUNK // 16):
        ones_v[pl.ds(i * 16, 16)] = jnp.full((16,), 1.0, _f32)

    @pl.when(sid == 0)
    def _():
        pltpu.sync_copy(zeros_hbm, acc)

    pltpu.sync_copy(dst_hbm.at[wid], didx)
    plsc.subcore_barrier()

    def body(i, carry):
        pltpu.sync_copy(ones_v, acc.at[didx.at[i]], add=True)
        return carry

    lax.fori_loop(0, DEG_ITERS, body, 0)
    plsc.subcore_barrier()

    @pl.when(sid == 0)
    def _():
        pltpu.sync_copy(acc, deg_out.at[cid])


def _sc_deg(dst3, zeros_n):
    mesh = plsc.VectorSubcoreMesh(core_axis_name="c", subcore_axis_name="s")
    return pl.kernel(
        _deg_body,
        mesh=mesh,
        out_type=jax.ShapeDtypeStruct((NC, N), _f32),
        scratch_types=[
            pltpu.VMEM((DEG_ITERS, CHUNK), jnp.int32),
            pltpu.VMEM((CHUNK,), _f32),
            pltpu.VMEM_SHARED((N,), _f32),
            pltpu.SemaphoreType.DMA,
        ],
    )(dst3, zeros_n)


SUP = 10
NSUP = AGG_ITERS // SUP


def _agg_body(src_hbm, dst_hbm, vtab_hbm, out_hbm,
              sidx, didx, gidx, rows0, rows1, acc, sem0, sem1):
    cid = lax.axis_index("c")
    sid = lax.axis_index("s")
    rows = (rows0, rows1)
    sems = (sem0, sem1)

    for h in range(2):
        off = h * 2 * N + cid * N
        pltpu.sync_copy(vtab_hbm.at[pl.ds(off + sid * NPT, NPT)],
                        acc.at[pl.ds(sid * NPT, NPT)])
        plsc.subcore_barrier()

        def sblock(s, carry):
            pltpu.sync_copy(src_hbm.at[sid, pl.ds(s * SUP, SUP)], sidx)
            pltpu.sync_copy(dst_hbm.at[sid, pl.ds(s * SUP, SUP)], didx)
            for b in range(SUP):
                for j in range(CHUNK // 16):
                    sl = pl.ds(j * 16, 16)
                    gidx[b, sl] = sidx[b, sl] + off
            g = [None, None]
            g[0] = pltpu.async_copy(vtab_hbm.at[gidx.at[0]], rows[0], sems[0])
            for b in range(SUP):
                j = b % 2
                nj = (b + 1) % 2
                if b + 1 < SUP:
                    g[nj] = pltpu.async_copy(vtab_hbm.at[gidx.at[b + 1]],
                                             rows[nj], sems[nj])
                g[j].wait()
                pltpu.sync_copy(rows[j], acc.at[didx.at[b]], add=True)
            return carry

        lax.fori_loop(0, NSUP, sblock, 0)
        plsc.subcore_barrier()
        pltpu.sync_copy(acc.at[pl.ds(sid * NPT, NPT)],
                        out_hbm.at[cid, h, pl.ds(sid * NPT, NPT)])
        plsc.subcore_barrier()


def _sc_agg(src3, dst3, vtab):
    mesh = plsc.VectorSubcoreMesh(core_axis_name="c", subcore_axis_name="s")
    return pl.kernel(
        _agg_body,
        mesh=mesh,
        compiler_params=pltpu.CompilerParams(use_tc_tiling_on_sc=False),
        out_type=jax.ShapeDtypeStruct((NC, 2, N, FH), _f32),
        scratch_types=[
            pltpu.VMEM((SUP, CHUNK), jnp.int32),
            pltpu.VMEM((SUP, CHUNK), jnp.int32),
            pltpu.VMEM((SUP, CHUNK), jnp.int32),
            pltpu.VMEM((CHUNK, FH), _f32),
            pltpu.VMEM((CHUNK, FH), _f32),
            pltpu.VMEM_SHARED((N, FH), _f32),
            pltpu.SemaphoreType.DMA,
            pltpu.SemaphoreType.DMA,
        ],
    )(src3, dst3, vtab)


_RT = 200


def _mm_kernel(x_ref, w_ref, d0_ref, d1_ref, out_ref):
    dinv = lax.rsqrt(d0_ref[...] + d1_ref[...] + 1.0)
    xb = x_ref[0]
    for p in range(P):
        xp = xb[:, :, p]
        u = jnp.dot(xp, w_ref[...], preferred_element_type=_f32)
        h, q = divmod(p, 6)
        out_ref[h, :, q * 24:(q + 1) * 24] = u * dinv


def _tc_matmul(x4, acat, d0, d1):
    nrt = N // _RT
    return pl.pallas_call(
        _mm_kernel,
        grid=(2 * nrt,),
        in_specs=[
            pl.BlockSpec((1, _RT, F_IN, P), lambda i: (i // nrt, i % nrt, 0, 0)),
            pl.BlockSpec((F_IN, 24), lambda i: (0, 0)),
            pl.BlockSpec((_RT, 1), lambda i: (i % nrt, 0)),
            pl.BlockSpec((_RT, 1), lambda i: (i % nrt, 0)),
        ],
        out_specs=pl.BlockSpec((2, _RT, FH), lambda i: (0, i, 0)),
        out_shape=jax.ShapeDtypeStruct((2, 2 * N, FH), _f32),
    )(x4, acat, d0, d1)


_NT = 1000


def _fin_kernel(agg_ref, x0_ref, d0_ref, d1_ref, bias_ref, wsel_ref,
                ow_ref, ob_ref, out_ref):
    dinv = lax.rsqrt(d0_ref[...] + d1_ref[...] + 1.0)
    acc = jnp.zeros((_NT, P), _f32)
    for h in range(2):
        u = agg_ref[0, h] * dinv + bias_ref[...]
        sig = jax.nn.sigmoid(u)
        th = jnp.tanh(u)
        thr = jnp.concatenate([th[:, 12:], th[:, :12]], axis=1)
        prod = (1.0 - sig) * thr
        acc = acc + jnp.dot(prod, wsel_ref[h], preferred_element_type=_f32)
    hrelu = jnp.maximum(acc + x0_ref[0], 0.0)
    out_ref[0] = jnp.dot(hrelu, ow_ref[...],
                         preferred_element_type=_f32) + ob_ref[...]


def _tc_final(agg, x0, d0, d1, bias144, wsel, ow, ob):
    return pl.pallas_call(
        _fin_kernel,
        grid=(2, N // _NT),
        in_specs=[
            pl.BlockSpec((1, 2, _NT, FH), lambda b, j: (b, 0, j, 0)),
            pl.BlockSpec((1, _NT, P), lambda b, j: (b, j, 0)),
            pl.BlockSpec((_NT, 1), lambda b, j: (j, 0)),
            pl.BlockSpec((_NT, 1), lambda b, j: (j, 0)),
            pl.BlockSpec((1, FH), lambda b, j: (0, 0)),
            pl.BlockSpec((2, FH, P), lambda b, j: (0, 0, 0)),
            pl.BlockSpec((P, P), lambda b, j: (0, 0)),
            pl.BlockSpec((1, P), lambda b, j: (0, 0)),
        ],
        out_specs=pl.BlockSpec((1, _NT, P), lambda b, j: (b, j, 0)),
        out_shape=jax.ShapeDtypeStruct((2, N, P), _f32),
    )(agg, x0, d0, d1, bias144, wsel, ow, ob)


def kernel(x, edge_index, attention, conv_z_W, conv_z_b, conv_r_W, conv_r_b,
           conv_h_W, conv_h_b, lin_z_W, lin_z_b, lin_r_W, lin_r_b,
           lin_h_W, lin_h_b, out_W, out_b):
    src = edge_index[0]
    dst = edge_index[1]
    dst_deg = dst.reshape(NC * NS, DEG_ITERS, CHUNK)
    src3 = src.reshape(NS, AGG_ITERS, CHUNK)
    dst3 = dst.reshape(NS, AGG_ITERS, CHUNK)

    deg2 = _sc_deg(dst_deg, jnp.zeros((N,), _f32))
    d0 = deg2[0].reshape(N, 1)
    d1 = deg2[1].reshape(N, 1)

    Az = conv_z_W @ lin_z_W[:P]
    Ah = conv_h_W @ lin_h_W[:P]
    Acat = jnp.concatenate([Az, Ah], axis=1)

    vtab = _tc_matmul(x, Acat, d0, d1).reshape(4 * N, FH)
    agg = _sc_agg(src3, dst3, vtab)

    x0 = x[:, :, 0, :]
    probs = jax.nn.softmax(attention)
    czc = conv_z_b @ lin_z_W[:P] + lin_z_b
    chc = conv_h_b @ lin_h_W[:P] + lin_h_b
    bias144 = jnp.tile(jnp.concatenate([czc, chc]), 6).reshape(1, FH)
    base = jnp.concatenate([jnp.eye(P, dtype=_f32),
                            jnp.zeros((P, P), _f32)], axis=0)
    wsel = (probs.reshape(2, 6)[:, :, None, None] *
            base[None, None]).reshape(2, FH, P)
    return _tc_final(agg, x0, d0, d1, bias144, wsel, out_W,
                     out_b.reshape(1, P))


# speedup vs baseline: 2.5862x; 2.5862x over previous
import functools

import jax
import jax.numpy as jnp
from jax import lax
from jax.experimental import pallas as pl
from jax.experimental.pallas import tpu as pltpu
from jax.experimental.pallas import tpu_sc as plsc

N = 10000
E = 320000
F_IN = 128
P = 12
NC = 2
NS = 16
CHUNK = 80
DEG_ITERS = E // (NC * NS) // CHUNK
AGG_ITERS = E // NS // CHUNK
NPT = N // NS
FH = 144

_f32 = jnp.float32


def _deg_body(dst_hbm, zeros_hbm, deg_out, didx, ones_v, acc, sem):
    cid = lax.axis_index("c")
    sid = lax.axis_index("s")
    wid = sid * NC + cid
    for i in range(CHUNK // 16):
        ones_v[pl.ds(i * 16, 16)] = jnp.full((16,), 1.0, _f32)

    @pl.when(sid == 0)
    def _():
        pltpu.sync_copy(zeros_hbm, acc)

    pltpu.sync_copy(dst_hbm.at[wid], didx)
    plsc.subcore_barrier()

    def body(i, carry):
        pltpu.sync_copy(ones_v, acc.at[didx.at[i]], add=True)
        return carry

    lax.fori_loop(0, DEG_ITERS, body, 0)
    plsc.subcore_barrier()

    @pl.when(sid == 0)
    def _():
        pltpu.sync_copy(acc, deg_out.at[cid])


def _sc_deg(dst3, zeros_n):
    mesh = plsc.VectorSubcoreMesh(core_axis_name="c", subcore_axis_name="s")
    return pl.kernel(
        _deg_body,
        mesh=mesh,
        out_type=jax.ShapeDtypeStruct((NC, N), _f32),
        scratch_types=[
            pltpu.VMEM((DEG_ITERS, CHUNK), jnp.int32),
            pltpu.VMEM((CHUNK,), _f32),
            pltpu.VMEM_SHARED((N,), _f32),
            pltpu.SemaphoreType.DMA,
        ],
    )(dst3, zeros_n)


SUP = 10
NSUP = AGG_ITERS // SUP


def _agg_body(src_hbm, dst_hbm, vtab_hbm, out_hbm,
              sidx, didx, gidx, rows0, rows1, acc, sem0, sem1):
    cid = lax.axis_index("c")
    sid = lax.axis_index("s")
    rows = (rows0, rows1)
    sems = (sem0, sem1)

    for h in range(2):
        off = (cid * 2 + h) * N
        pltpu.sync_copy(vtab_hbm.at[pl.ds(off + sid * NPT, NPT)],
                        acc.at[pl.ds(sid * NPT, NPT)])
        plsc.subcore_barrier()

        def sblock(s, carry):
            pltpu.sync_copy(src_hbm.at[sid, pl.ds(s * SUP, SUP)], sidx)
            pltpu.sync_copy(dst_hbm.at[sid, pl.ds(s * SUP, SUP)], didx)
            for b in range(SUP):
                for j in range(CHUNK // 16):
                    sl = pl.ds(j * 16, 16)
                    gidx[b, sl] = sidx[b, sl] + off
            g = [None, None]
            g[0] = pltpu.async_copy(vtab_hbm.at[gidx.at[0]], rows[0], sems[0])
            for b in range(SUP):
                j = b % 2
                nj = (b + 1) % 2
                if b + 1 < SUP:
                    g[nj] = pltpu.async_copy(vtab_hbm.at[gidx.at[b + 1]],
                                             rows[nj], sems[nj])
                g[j].wait()
                pltpu.sync_copy(rows[j], acc.at[didx.at[b]], add=True)
            return carry

        lax.fori_loop(0, NSUP, sblock, 0)
        plsc.subcore_barrier()
        pltpu.sync_copy(acc.at[pl.ds(sid * NPT, NPT)],
                        out_hbm.at[cid, h, pl.ds(sid * NPT, NPT)])
        plsc.subcore_barrier()


def _sc_agg(src3, dst3, vtab):
    mesh = plsc.VectorSubcoreMesh(core_axis_name="c", subcore_axis_name="s")
    return pl.kernel(
        _agg_body,
        mesh=mesh,
        compiler_params=pltpu.CompilerParams(use_tc_tiling_on_sc=False),
        out_type=jax.ShapeDtypeStruct((NC, 2, N, FH), _f32),
        scratch_types=[
            pltpu.VMEM((SUP, CHUNK), jnp.int32),
            pltpu.VMEM((SUP, CHUNK), jnp.int32),
            pltpu.VMEM((SUP, CHUNK), jnp.int32),
            pltpu.VMEM((CHUNK, FH), _f32),
            pltpu.VMEM((CHUNK, FH), _f32),
            pltpu.VMEM_SHARED((N, FH), _f32),
            pltpu.SemaphoreType.DMA,
            pltpu.SemaphoreType.DMA,
        ],
    )(src3, dst3, vtab)


_RT = 200


def _mm_kernel(x_ref, w_ref, d0_ref, d1_ref, out_ref):
    dinv = lax.rsqrt(d0_ref[...] + d1_ref[...] + 1.0)
    u = jnp.dot(x_ref[...], w_ref[0], preferred_element_type=_f32)
    out_ref[...] = u * dinv


def _tc_matmul(xr, w_bd, d0, d1):
    nrt = N // _RT
    return pl.pallas_call(
        _mm_kernel,
        grid=(2 * nrt, 2),
        in_specs=[
            pl.BlockSpec((_RT, F_IN * P), lambda i, h: (i, 0)),
            pl.BlockSpec((1, F_IN * P, FH), lambda i, h: (h, 0, 0)),
            pl.BlockSpec((_RT, 1), lambda i, h: (i % nrt, 0)),
            pl.BlockSpec((_RT, 1), lambda i, h: (i % nrt, 0)),
        ],
        out_specs=pl.BlockSpec(
            (_RT, FH), lambda i, h: ((i // nrt) * 2 * nrt + h * nrt + i % nrt, 0)),
        out_shape=jax.ShapeDtypeStruct((2 * 2 * N, FH), _f32),
    )(xr, w_bd, d0, d1)


_NT = 1000


def _fin_kernel(agg_ref, x0_ref, d0_ref, d1_ref, bias_ref, wsel_ref,
                ow_ref, ob_ref, out_ref):
    dinv = lax.rsqrt(d0_ref[...] + d1_ref[...] + 1.0)
    acc = jnp.zeros((_NT, P), _f32)
    for h in range(2):
        u = agg_ref[0, h] * dinv + bias_ref[...]
        sig = jax.nn.sigmoid(u)
        th = jnp.tanh(u)
        thr = jnp.concatenate([th[:, 12:], th[:, :12]], axis=1)
        prod = (1.0 - sig) * thr
        acc = acc + jnp.dot(prod, wsel_ref[h], preferred_element_type=_f32)
    hrelu = jnp.maximum(acc + x0_ref[0], 0.0)
    out_ref[0] = jnp.dot(hrelu, ow_ref[...],
                         preferred_element_type=_f32) + ob_ref[...]


def _tc_final(agg, x0, d0, d1, bias144, wsel, ow, ob):
    return pl.pallas_call(
        _fin_kernel,
        grid=(2, N // _NT),
        in_specs=[
            pl.BlockSpec((1, 2, _NT, FH), lambda b, j: (b, 0, j, 0)),
            pl.BlockSpec((1, _NT, P), lambda b, j: (b, j, 0)),
            pl.BlockSpec((_NT, 1), lambda b, j: (j, 0)),
            pl.BlockSpec((_NT, 1), lambda b, j: (j, 0)),
            pl.BlockSpec((1, FH), lambda b, j: (0, 0)),
            pl.BlockSpec((2, FH, P), lambda b, j: (0, 0, 0)),
            pl.BlockSpec((P, P), lambda b, j: (0, 0)),
            pl.BlockSpec((1, P), lambda b, j: (0, 0)),
        ],
        out_specs=pl.BlockSpec((1, _NT, P), lambda b, j: (b, j, 0)),
        out_shape=jax.ShapeDtypeStruct((2, N, P), _f32),
    )(agg, x0, d0, d1, bias144, wsel, ow, ob)


def kernel(x, edge_index, attention, conv_z_W, conv_z_b, conv_r_W, conv_r_b,
           conv_h_W, conv_h_b, lin_z_W, lin_z_b, lin_r_W, lin_r_b,
           lin_h_W, lin_h_b, out_W, out_b):
    src = edge_index[0]
    dst = edge_index[1]
    dst_deg = dst.reshape(NC * NS, DEG_ITERS, CHUNK)
    src3 = src.reshape(NS, AGG_ITERS, CHUNK)
    dst3 = dst.reshape(NS, AGG_ITERS, CHUNK)

    deg2 = _sc_deg(dst_deg, jnp.zeros((N,), _f32))
    d0 = deg2[0].reshape(N, 1)
    d1 = deg2[1].reshape(N, 1)

    Az = conv_z_W @ lin_z_W[:P]
    Ah = conv_h_W @ lin_h_W[:P]
    Acat = jnp.concatenate([Az, Ah], axis=1)
    w_bd = (Acat[:, None, None, :] *
            jnp.eye(P, dtype=_f32)[None, :, :, None]).reshape(F_IN * P, P * 24)
    w_bd = w_bd.reshape(F_IN * P, 2, FH).transpose(1, 0, 2)

    xr = x.reshape(2 * N, F_IN * P)
    vtab = _tc_matmul(xr, w_bd, d0, d1)
    agg = _sc_agg(src3, dst3, vtab)

    x0 = x[:, :, 0, :]
    probs = jax.nn.softmax(attention)
    czc = conv_z_b @ lin_z_W[:P] + lin_z_b
    chc = conv_h_b @ lin_h_W[:P] + lin_h_b
    bias144 = jnp.tile(jnp.concatenate([czc, chc]), 6).reshape(1, FH)
    base = jnp.concatenate([jnp.eye(P, dtype=_f32),
                            jnp.zeros((P, P), _f32)], axis=0)
    wsel = (probs.reshape(2, 6)[:, :, None, None] *
            base[None, None]).reshape(2, FH, P)
    return _tc_final(agg, x0, d0, d1, bias144, wsel, out_W,
                     out_b.reshape(1, P))
